# Initial kernel scaffold; baseline (speedup 1.0000x reference)
import jax
import jax.numpy as jnp
from jax.experimental import pallas as pl


def kernel(xyz1, xyz2, points1, points2, wn_w1, wn_b1, wn_g1, wn_be1, wn_w2, wn_b2, wn_g2, wn_be2, wn_w3, wn_b3, wn_g3, wn_be3, dn_w1, dn_b1, dn_g1, dn_be1, dn_w2, dn_b2, dn_g2, dn_be2, dn_w3, dn_b3, dn_g3, dn_be3, lin_w, lin_b, bn_g, bn_b):
    raise NotImplementedError("write your pallas kernel here")


# speedup vs baseline: 4.9085x; 4.9085x over previous
import functools

import jax
import jax.numpy as jnp
from jax import lax
from jax.experimental import pallas as pl
from jax.experimental.pallas import tpu as pltpu
from jax.experimental.pallas import tpu_sc as plsc

NSAMPLE = 16
BW = 0.5
EPS_BN = 1e-5
TQ = 256
TP = 512
TP6 = 256
TN7 = 512
F32 = jnp.float32


def _stage1_body(x1_ref, x1t_ref, x2_ref, p2_ref,
                 src_main_ref, invd_tab_ref, gidx_ref):
    b = pl.program_id(0)
    x1 = x1_ref[0]
    qt = x1t_ref[0]
    x2 = x2_ref[0]
    p2 = p2_ref[0]
    n_all = x1.shape[1]
    s_all = x2.shape[1]

    qn = jnp.sum(qt * qt, axis=1)[:, None]

    kn2 = jnp.sum(x2 * x2, axis=0)[None, :]
    d2a = (-2.0 * jnp.dot(qt, x2) + qn) + kn2
    cols_s = lax.broadcasted_iota(jnp.int32, (TQ, s_all), 1)
    d = d2a
    masks = []
    dmins = []
    for _ in range(3):
        m = jnp.min(d, axis=1)
        amin = jnp.min(jnp.where(d == m[:, None], cols_s, s_all), axis=1)
        msk = cols_s == amin[:, None]
        masks.append(msk)
        dmins.append(m)
        d = jnp.where(msk, jnp.float32(jnp.inf), d)
    recips = [1.0 / (m + 1e-8) for m in dmins]
    norm = recips[0] + recips[1] + recips[2]
    wmat = ((recips[0] / norm)[:, None] * masks[0].astype(F32)
            + (recips[1] / norm)[:, None] * masks[1].astype(F32)
            + (recips[2] / norm)[:, None] * masks[2].astype(F32))
    interp = jnp.dot(wmat, p2, precision=lax.Precision.HIGHEST)

    qpad = jnp.concatenate([qt, jnp.zeros((TQ, 125), F32)], axis=1)
    src_main_ref[...] = interp + qpad

    kn1 = jnp.sum(x1 * x1, axis=0)[None, :]
    d2b = (-2.0 * jnp.dot(qt, x1) + qn) + kn1
    gauss = jnp.exp(d2b * (-1.0 / (2.0 * BW * BW))) * (1.0 / (2.5 * BW))
    dens = jnp.mean(gauss, axis=1)
    invd = 1.0 / dens

    cols_n = lax.broadcasted_iota(jnp.int32, (TQ, n_all), 1)
    d = d2b
    idx_cols = []
    for _ in range(NSAMPLE):
        m = jnp.min(d, axis=1)
        amin = jnp.min(jnp.where(d == m[:, None], cols_n, n_all), axis=1)
        idx_cols.append(amin[:, None])
        d = jnp.where(cols_n == amin[:, None], jnp.float32(jnp.inf), d)
    gidx = jnp.concatenate(idx_cols, axis=1)
    gidx_ref[...] = gidx + b * n_all

    invd_tab_ref[...] = invd.reshape(1, TQ // 128, 128)


def _stage1(xyz1, x1t, xyz2, p2pad):
    B, _, N = xyz1.shape
    S = xyz2.shape[2]
    nt = N // TQ
    return pl.pallas_call(
        _stage1_body,
        grid=(B, nt),
        in_specs=[
            pl.BlockSpec((1, 3, N), lambda b, t: (b, 0, 0)),
            pl.BlockSpec((1, TQ, 3), lambda b, t: (b, t, 0)),
            pl.BlockSpec((1, 3, S), lambda b, t: (b, 0, 0)),
            pl.BlockSpec((1, S, 128), lambda b, t: (b, 0, 0)),
        ],
        out_specs=[
            pl.BlockSpec((TQ, 128), lambda b, t, _n=nt: (b * _n + t, 0)),
            pl.BlockSpec((1, TQ // 128, 128),
                         lambda b, t, _n=nt: (b * _n + t, 0, 0)),
            pl.BlockSpec((TQ, 16), lambda b, t, _n=nt: (b * _n + t, 0)),
        ],
        out_shape=[
            jax.ShapeDtypeStruct((B * N, 128), F32),
            jax.ShapeDtypeStruct((B * N // TQ, TQ // 128, 128), F32),
            jax.ShapeDtypeStruct((B * N, 16), jnp.int32),
        ],
    )(xyz1, x1t, xyz2, p2pad)


def _make_sc_gather(total_rows, n_src):
    NC, NS = 2, 16
    NW = NC * NS
    rows_w = total_rows // NW
    ch = rows_w // 128
    ng = rows_w // 16
    mesh = plsc.VectorSubcoreMesh(core_axis_name="c", subcore_axis_name="s")

    @functools.partial(
        pl.kernel, mesh=mesh,
        compiler_params=pltpu.CompilerParams(needs_layout_passes=False),
        out_type=[
            jax.ShapeDtypeStruct((total_rows, 128), F32),
            jax.ShapeDtypeStruct((total_rows,), F32),
        ],
        scratch_types=[
            pltpu.VMEM((ch, 128), jnp.int32),
            pltpu.VMEM((ng, 16), jnp.int32),
            pltpu.VMEM((n_src,), F32),
            pltpu.VMEM((128, 128), F32),
            pltpu.VMEM((rows_w,), F32),
            pltpu.SemaphoreType.DMA,
        ],
    )
    def sc_gather(src_main, invd_tab, idx, idx2, out_main, out_invd,
                  idx_v, idx_v2, invd_v, bufm, invd_out, semm):
        wid = lax.axis_index("s") * NC + lax.axis_index("c")
        pltpu.sync_copy(idx.at[wid], idx_v)
        pltpu.sync_copy(idx2.at[wid], idx_v2)
        pltpu.sync_copy(invd_tab, invd_v)

        def body(j, carry):
            cm = pltpu.make_async_copy(src_main.at[idx_v.at[j]], bufm, semm)
            cm.start()
            cm.wait()
            pltpu.sync_copy(bufm,
                            out_main.at[pl.ds(wid * rows_w + j * 128, 128)])
            return carry

        lax.fori_loop(0, ch, body, 0)

        def gbody(i, carry):
            idxvec = idx_v2[i]
            invd_out[pl.ds(i * 16, 16)] = plsc.load_gather(invd_v, [idxvec])
            return carry

        lax.fori_loop(0, ng, gbody, 0)
        pltpu.sync_copy(invd_out, out_invd.at[pl.ds(wid * rows_w, rows_w)])

    return sc_gather


def _group_prefix(main_ref, invd_ref, x1f_ref):
    mn = main_ref[...]
    xyz = x1f_ref[...]
    tp = mn.shape[0]
    r = tp * 16
    xyz_rep = jnp.broadcast_to(xyz[:, None, :], (tp, 16, 3)).reshape(r, 3)
    g = mn[:, :, 0:3].reshape(r, 3) - xyz_rep
    invd = invd_ref[...]
    invmax = jnp.max(invd, axis=1, keepdims=True)
    ds0 = (invd / invmax).reshape(r, 1)
    return g, ds0, r


def _moment_matrix(ext, ncols, used):
    rows = []
    for c in range(ncols):
        if c < used:
            rows.append(jnp.sum(ext * ext[:, c:c + 1], axis=0, keepdims=True))
        else:
            rows.append(jnp.zeros((1, ncols), F32))
    return jnp.concatenate(rows, axis=0)


def _accum_out(out_ref, val):
    @pl.when(pl.program_id(0) == 0)
    def _():
        out_ref[...] = jnp.zeros_like(out_ref)
    out_ref[...] += val


def _stats0_body(main_ref, invd_ref, x1f_ref, out_ref):
    g, ds0, r = _group_prefix(main_ref, invd_ref, x1f_ref)
    ext = jnp.concatenate(
        [g, ds0, jnp.ones((r, 1), F32), jnp.zeros((r, 3), F32)], axis=1)
    _accum_out(out_ref, _moment_matrix(ext, 8, 5))


def _layer1(g, ds0, w1wt_ref, b1w_ref, w1d_ref, b1d_ref):
    a1w = jax.nn.relu(jnp.dot(g, w1wt_ref[...], precision=lax.Precision.HIGHEST) + b1w_ref[...])
    a1d = jax.nn.relu(ds0 * w1d_ref[...] + b1d_ref[...])
    return a1w, a1d


def _stats1_body(main_ref, invd_ref, x1f_ref, w1wt_ref, b1w_ref, w1d_ref,
                 b1d_ref, out_ref):
    g, ds0, r = _group_prefix(main_ref, invd_ref, x1f_ref)
    a1w, a1d = _layer1(g, ds0, w1wt_ref, b1w_ref, w1d_ref, b1d_ref)
    ext = jnp.concatenate(
        [a1d, a1w, jnp.ones((r, 1), F32), jnp.zeros((r, 7), F32)], axis=1)
    _accum_out(out_ref, _moment_matrix(ext, 32, 25))


def _layer2(a1w, a1d, w2wt_ref, b2w_ref, w2dt_ref, b2d_ref):
    a2w = jax.nn.relu(jnp.dot(a1w, w2wt_ref[...], precision=lax.Precision.HIGHEST) + b2w_ref[...])
    a2d = jax.nn.relu(jnp.dot(a1d, w2dt_ref[...], precision=lax.Precision.HIGHEST) + b2d_ref[...])
    return a2w, a2d


def _stats2_body(main_ref, invd_ref, x1f_ref, w1wt_ref, b1w_ref, w1d_ref,
                 b1d_ref, w2wt_ref, b2w_ref, w2dt_ref, b2d_ref, out_ref):
    g, ds0, r = _group_prefix(main_ref, invd_ref, x1f_ref)
    a1w, a1d = _layer1(g, ds0, w1wt_ref, b1w_ref, w1d_ref, b1d_ref)
    a2w, a2d = _layer2(a1w, a1d, w2wt_ref, b2w_ref, w2dt_ref, b2d_ref)
    ext = jnp.concatenate(
        [a2d, a2w, jnp.ones((r, 1), F32), jnp.zeros((r, 15), F32)], axis=1)
    _accum_out(out_ref, _moment_matrix(ext, 32, 17))


def _param_spec(shape):
    return pl.BlockSpec(shape, lambda t: tuple(0 for _ in shape))


def _stats_call(body, main3, invd3, x1f, params, out_dim):
    nblk = main3.shape[0] // TP
    in_specs = [
        pl.BlockSpec((TP, 16, 128), lambda t: (t, 0, 0)),
        pl.BlockSpec((TP, 16, 1), lambda t: (t, 0, 0)),
        pl.BlockSpec((TP, 3), lambda t: (t, 0)),
    ] + [_param_spec(p.shape) for p in params]
    return pl.pallas_call(
        body,
        grid=(nblk,),
        in_specs=in_specs,
        out_specs=pl.BlockSpec((out_dim, out_dim), lambda t: (0, 0)),
        out_shape=jax.ShapeDtypeStruct((out_dim, out_dim), F32),
    )(main3, invd3, x1f, *params)


def _stage6_body(main_ref, invd_ref, x1f_ref,
                 w1wt_ref, b1w_ref, w1d_ref, b1d_ref,
                 w2wt_ref, b2w_ref, w2dt_ref, b2d_ref,
                 w3wt_ref, b3w_ref, w3d_ref, b3d_ref,
                 linp_ref, linb_ref, ypre_ref, mom_ref):
    g, ds0, r = _group_prefix(main_ref, invd_ref, x1f_ref)
    a1w, a1d = _layer1(g, ds0, w1wt_ref, b1w_ref, w1d_ref, b1d_ref)
    a2w, a2d = _layer2(a1w, a1d, w2wt_ref, b2w_ref, w2dt_ref, b2d_ref)
    w = jax.nn.relu(jnp.dot(a2w, w3wt_ref[...], precision=lax.Precision.HIGHEST) + b3w_ref[...])
    dsc = jax.nn.relu(
        jnp.sum(a2d * w3d_ref[...], axis=1, keepdims=True) + b3d_ref[...])
    wd = w * dsc
    wd3 = wd.reshape(TP6, 16, 16)

    xyz = x1f_ref[...]
    centerpad = jnp.concatenate([xyz, jnp.zeros((TP6, 125), F32)], axis=1)
    npr = main_ref[...] - centerpad[:, None, :]

    us = []
    for j in range(16):
        uj = wd3[:, 0, j:j + 1] * npr[:, 0, :]
        for k in range(1, 16):
            uj = uj + wd3[:, k, j:j + 1] * npr[:, k, :]
        us.append(uj)
    ucat = jnp.concatenate(us, axis=1)
    y = jnp.dot(ucat, linp_ref[...], precision=lax.Precision.HIGHEST) + linb_ref[...]
    ypre_ref[...] = y

    rowi = lax.broadcasted_iota(jnp.int32, (8, 128), 0)
    part = (jnp.where(rowi == 0, jnp.sum(y, axis=0)[None, :], 0.0)
            + jnp.where(rowi == 1, jnp.sum(y * y, axis=0)[None, :], 0.0))
    @pl.when(pl.program_id(0) == 0)
    def _():
        mom_ref[...] = jnp.zeros_like(mom_ref)
    mom_ref[...] += part


def _stage6(main3, invd3, x1f, params):
    nblk = main3.shape[0] // TP6
    in_specs = [
        pl.BlockSpec((TP6, 16, 128), lambda t: (t, 0, 0)),
        pl.BlockSpec((TP6, 16, 1), lambda t: (t, 0, 0)),
        pl.BlockSpec((TP6, 3), lambda t: (t, 0)),
    ] + [_param_spec(p.shape) for p in params]
    return pl.pallas_call(
        _stage6_body,
        grid=(nblk,),
        in_specs=in_specs,
        out_specs=[
            pl.BlockSpec((TP6, 128), lambda t: (t, 0)),
            pl.BlockSpec((8, 128), lambda t: (0, 0)),
        ],
        out_shape=[
            jax.ShapeDtypeStruct((main3.shape[0], 128), F32),
            jax.ShapeDtypeStruct((8, 128), F32),
        ],
    )(main3, invd3, x1f, *params)


def _stage7_body(y_ref, a_ref, c_ref, out_ref):
    y = y_ref[0]
    z = jax.nn.relu(y * a_ref[...] + c_ref[...])
    out_ref[0] = z.T


def _stage7(ypre3, aff_a, aff_c):
    B, N, _ = ypre3.shape
    nt = N // TN7
    return pl.pallas_call(
        _stage7_body,
        grid=(B, nt),
        in_specs=[
            pl.BlockSpec((1, TN7, 128), lambda b, t: (b, t, 0)),
            pl.BlockSpec((1, 128), lambda b, t: (0, 0)),
            pl.BlockSpec((1, 128), lambda b, t: (0, 0)),
        ],
        out_specs=pl.BlockSpec((1, 128, TN7), lambda b, t: (b, 0, t)),
        out_shape=jax.ShapeDtypeStruct((B, 128, N), F32),
    )(ypre3, aff_a, aff_c)


def _bn_fold(w, b, g, beta, m1, m2):
    hp = lax.Precision.HIGHEST
    wm1 = jnp.dot(w, m1, precision=hp)
    mean = wm1 + b
    ey2 = jnp.einsum('oi,ij,oj->o', w, m2, w, precision=hp) \
        + 2.0 * b * wm1 + b * b
    var = ey2 - mean * mean
    a = g * lax.rsqrt(var + EPS_BN)
    c = beta - mean * a
    return a[:, None] * w, a * b + c


def kernel(xyz1, xyz2, points1, points2,
           wn_w1, wn_b1, wn_g1, wn_be1,
           wn_w2, wn_b2, wn_g2, wn_be2,
           wn_w3, wn_b3, wn_g3, wn_be3,
           dn_w1, dn_b1, dn_g1, dn_be1,
           dn_w2, dn_b2, dn_g2, dn_be2,
           dn_w3, dn_b3, dn_g3, dn_be3,
           lin_w, lin_b, bn_g, bn_b):
    del points1
    B, _, N = xyz1.shape
    S = xyz2.shape[2]
    BN = B * N
    cnt = jnp.float32(BN * NSAMPLE)

    x1t = jnp.transpose(xyz1, (0, 2, 1))
    p2pad = jnp.pad(jnp.transpose(points2, (0, 2, 1)),
                    ((0, 0), (0, 0), (3, 0)))

    src_main, invd_tab, gidx = _stage1(xyz1, x1t, xyz2, p2pad)

    total_rows = BN * NSAMPLE
    idx_r = gidx.reshape(32, total_rows // 32 // 128, 128)
    idx_r2 = gidx.reshape(32, total_rows // 32 // 16, 16)
    out_main, out_invd = _make_sc_gather(total_rows, BN)(
        src_main, invd_tab.reshape(BN), idx_r, idx_r2)
    invd3 = out_invd.reshape(BN, 16, 1)
    main3 = out_main.reshape(BN, 16, 128)
    x1f = x1t.reshape(BN, 3)

    s0 = _stats_call(_stats0_body, main3, invd3, x1f, [], 8)
    m1g = s0[0:3, 4] / cnt
    m2g = s0[0:3, 0:3] / cnt
    m1d = s0[3:4, 4] / cnt
    m2d = s0[3:4, 3:4] / cnt
    w1w_eff, b1w_eff = _bn_fold(wn_w1, wn_b1, wn_g1, wn_be1, m1g, m2g)
    w1d_eff, b1d_eff = _bn_fold(dn_w1, dn_b1, dn_g1, dn_be1, m1d, m2d)
    p_l1 = [w1w_eff.T, b1w_eff[None, :], w1d_eff[:, 0][None, :],
            b1d_eff[None, :]]

    s1 = _stats_call(_stats1_body, main3, invd3, x1f, p_l1, 32)
    m1a1d = s1[0:16, 24] / cnt
    m2a1d = s1[0:16, 0:16] / cnt
    m1a1w = s1[16:24, 24] / cnt
    m2a1w = s1[16:24, 16:24] / cnt
    w2w_eff, b2w_eff = _bn_fold(wn_w2, wn_b2, wn_g2, wn_be2, m1a1w, m2a1w)
    w2d_eff, b2d_eff = _bn_fold(dn_w2, dn_b2, dn_g2, dn_be2, m1a1d, m2a1d)
    p_l2 = p_l1 + [w2w_eff.T, b2w_eff[None, :], w2d_eff.T, b2d_eff[None, :]]

    s2 = _stats_call(_stats2_body, main3, invd3, x1f, p_l2, 32)
    m1a2d = s2[0:8, 16] / cnt
    m2a2d = s2[0:8, 0:8] / cnt
    m1a2w = s2[8:16, 16] / cnt
    m2a2w = s2[8:16, 8:16] / cnt
    w3w_eff, b3w_eff = _bn_fold(wn_w3, wn_b3, wn_g3, wn_be3, m1a2w, m2a2w)
    w3d_eff, b3d_eff = _bn_fold(dn_w3, dn_b3, dn_g3, dn_be3, m1a2d, m2a2d)

    linp = lin_w.reshape(128, 128, 16).transpose(2, 1, 0).reshape(2048, 128)
    p_l3 = p_l2 + [w3w_eff.T, b3w_eff[None, :], w3d_eff[0][None, :],
                   b3d_eff[None, :], linp, lin_b[None, :]]

    ypre, mom = _stage6(main3, invd3, x1f, p_l3)

    cnt_y = jnp.float32(BN)
    mean_y = mom[0] / cnt_y
    var_y = mom[1] / cnt_y - mean_y * mean_y
    aff_a = bn_g * lax.rsqrt(var_y + EPS_BN)
    aff_c = bn_b - mean_y * aff_a

    return _stage7(ypre.reshape(B, N, 128), aff_a[None, :], aff_c[None, :])
